# Initial kernel scaffold; baseline (speedup 1.0000x reference)
import jax
import jax.numpy as jnp
from jax.experimental import pallas as pl


def kernel(x, table):
    raise NotImplementedError("write your pallas kernel here")


# speedup vs baseline: 1.7967x; 1.7967x over previous
import functools

import jax
import jax.numpy as jnp
from jax import lax
from jax.experimental import pallas as pl
from jax.experimental.pallas import tpu as pltpu
from jax.experimental.pallas import tpu_sc as plsc

D_MODEL = 64
NC = 2
NS = 16
NW = NC * NS
B = 16384 * 50
B_PER_W = B // NW
CHUNK = 512
N_CHUNKS = B_PER_W // CHUNK

_mesh = plsc.VectorSubcoreMesh(core_axis_name="c", subcore_axis_name="s")


@functools.partial(
    pl.kernel,
    mesh=_mesh,
    out_type=jax.ShapeDtypeStruct((B, D_MODEL), jnp.float32),
    scratch_types=[
        pltpu.VMEM((CHUNK,), jnp.int32),
        pltpu.VMEM((CHUNK, D_MODEL), jnp.float32),
        pltpu.SemaphoreType.DMA,
    ],
    compiler_params=pltpu.CompilerParams(use_tc_tiling_on_sc=False),
)
def _gather_all(idx_hbm, table_hbm, out_hbm, idx_v, rows_v, sem):
    wid = lax.axis_index("s") * NC + lax.axis_index("c")
    base = wid * B_PER_W

    def body(g, carry):
        off = base + g * CHUNK
        pltpu.sync_copy(idx_hbm.at[pl.ds(off, CHUNK)], idx_v)
        pltpu.async_copy(table_hbm.at[idx_v], rows_v, sem).wait()
        pltpu.sync_copy(rows_v, out_hbm.at[pl.ds(off, CHUNK)])
        return carry

    lax.fori_loop(0, N_CHUNKS, body, 0)


def kernel(x, table):
    flat = x.reshape(-1).astype(jnp.int32)
    out = _gather_all(flat, table)
    return out.reshape(x.shape[0], x.shape[1], D_MODEL)


# speedup vs baseline: 1.8832x; 1.0482x over previous
import functools

import jax
import jax.numpy as jnp
from jax import lax
from jax.experimental import pallas as pl
from jax.experimental.pallas import tpu as pltpu
from jax.experimental.pallas import tpu_sc as plsc

D_MODEL = 64
NC = 2
NS = 16
NW = NC * NS
B = 16384 * 50
B_PER_W = B // NW
NBUF = 4
CHUNK = 320
ROUND = NBUF * CHUNK
N_ROUNDS = B_PER_W // ROUND
assert B_PER_W % ROUND == 0

_mesh = plsc.VectorSubcoreMesh(core_axis_name="c", subcore_axis_name="s")

_scratch = (
    [pltpu.VMEM((B_PER_W,), jnp.int32)]
    + [pltpu.VMEM((CHUNK, D_MODEL), jnp.float32) for _ in range(NBUF)]
    + [pltpu.SemaphoreType.DMA for _ in range(NBUF)]
    + [pltpu.SemaphoreType.DMA for _ in range(NBUF)]
)


@functools.partial(
    pl.kernel,
    mesh=_mesh,
    out_type=jax.ShapeDtypeStruct((B, D_MODEL), jnp.float32),
    scratch_types=_scratch,
    compiler_params=pltpu.CompilerParams(use_tc_tiling_on_sc=False),
)
def _gather_all(idx_hbm, table_hbm, out_hbm, idx_v, *bufs):
    rows = list(bufs[:NBUF])
    gsem = list(bufs[NBUF:2 * NBUF])
    osem = list(bufs[2 * NBUF:3 * NBUF])

    wid = lax.axis_index("s") * NC + lax.axis_index("c")
    base = wid * B_PER_W

    pltpu.sync_copy(idx_hbm.at[pl.ds(base, B_PER_W)], idx_v)

    def fire_gather(r, b):
        off = r * ROUND + b * CHUNK
        pltpu.make_async_copy(
            table_hbm.at[idx_v.at[pl.ds(off, CHUNK)]], rows[b], gsem[b]
        ).start()

    def wait_gather(r, b):
        off = r * ROUND + b * CHUNK
        pltpu.make_async_copy(
            table_hbm.at[idx_v.at[pl.ds(off, CHUNK)]], rows[b], gsem[b]
        ).wait()

    def fire_store(r, b):
        off = base + r * ROUND + b * CHUNK
        pltpu.make_async_copy(
            rows[b], out_hbm.at[pl.ds(off, CHUNK)], osem[b]
        ).start()

    def wait_store(r, b):
        off = base + r * ROUND + b * CHUNK
        pltpu.make_async_copy(
            rows[b], out_hbm.at[pl.ds(off, CHUNK)], osem[b]
        ).wait()

    for b in range(NBUF):
        fire_gather(0, b)

    def body(r, carry):
        for b in range(NBUF):
            wait_gather(r, b)
            fire_store(r, b)
        for b in range(NBUF):
            wait_store(r, b)
            fire_gather(r + 1, b)
        return carry

    lax.fori_loop(0, N_ROUNDS - 1, body, 0)

    r = N_ROUNDS - 1
    for b in range(NBUF):
        wait_gather(r, b)
        fire_store(r, b)
    for b in range(NBUF):
        wait_store(r, b)


def kernel(x, table):
    flat = x.reshape(-1).astype(jnp.int32)
    out = _gather_all(flat, table)
    return out.reshape(x.shape[0], x.shape[1], D_MODEL)
